# baseline (speedup 1.0000x reference)
import functools

import jax
import jax.numpy as jnp
from jax.experimental import pallas as pl
from jax.experimental.pallas import tpu as pltpu

_N = 10000
_BI = 200
_NI = _N // _BI


def _fused_kernel(seq_ref, w_ref, adj_ref, gamma_ref, beta_ref, pw_ref,
                  out_ref, xbf_ref, a0_ref, a1_ref, sum_ref, sq_ref):
    i = pl.program_id(0)
    parity = jax.lax.rem(i, 2)

    @pl.when(i == 0)
    def _compute_x():
        x = jax.lax.dot_general(
            seq_ref[...], w_ref[...],
            dimension_numbers=(((1,), (1,)), ((), ())),
            preferred_element_type=jnp.float32,
        )
        xbf_ref[...] = x.astype(jnp.bfloat16)

    @pl.when((i < _NI) & (parity == 0))
    def _cast_even():
        a0_ref[...] = adj_ref[...].astype(jnp.bfloat16)

    @pl.when((i < _NI) & (parity == 1))
    def _cast_odd():
        a1_ref[...] = adj_ref[...].astype(jnp.bfloat16)

    @pl.when(i > 0)
    def _matmul_prev():
        j = i - 1

        def _consume(a_ref):
            y_blk = jnp.dot(a_ref[...], xbf_ref[...],
                            preferred_element_type=jnp.float32)
            out_ref[pl.ds(j * _BI, _BI), :] = y_blk
            s = jnp.sum(y_blk, axis=0, keepdims=True)
            q = jnp.sum(y_blk * y_blk, axis=0, keepdims=True)

            @pl.when(j == 0)
            def _init_stats():
                sum_ref[...] = s
                sq_ref[...] = q

            @pl.when(j > 0)
            def _acc_stats():
                sum_ref[...] += s
                sq_ref[...] += q

        @pl.when(parity == 1)
        def _consume_even():
            _consume(a0_ref)

        @pl.when(parity == 0)
        def _consume_odd():
            _consume(a1_ref)

    @pl.when(i == _NI)
    def _epilogue():
        mean = sum_ref[...] / _N
        var = sq_ref[...] / _N - mean * mean
        scale = gamma_ref[...] / jnp.sqrt(var + 1e-5)
        o = (out_ref[...] - mean) * scale + beta_ref[...]
        out_ref[...] = jnp.where(o >= 0, o, pw_ref[...] * o)


@functools.partial(jax.jit, static_argnames=("interpret",))
def kernel(seq, adj, W, bias, gamma, beta, prelu_w, interpret=False):
    n, in_ft = seq.shape
    out_ft = W.shape[0]

    gamma2 = gamma.reshape(1, out_ft)
    beta2 = beta.reshape(1, out_ft)
    pw2 = jnp.broadcast_to(prelu_w.reshape(1, -1), (1, out_ft))
    del bias

    const = lambda i: (0, 0)
    out = pl.pallas_call(
        _fused_kernel,
        grid=(_NI + 1,),
        in_specs=[
            pl.BlockSpec((n, in_ft), const),
            pl.BlockSpec((out_ft, in_ft), const),
            pl.BlockSpec((_BI, n), lambda i: (jnp.minimum(i, _NI - 1), 0)),
            pl.BlockSpec((1, out_ft), const),
            pl.BlockSpec((1, out_ft), const),
            pl.BlockSpec((1, out_ft), const),
        ],
        out_specs=pl.BlockSpec((n, out_ft), const),
        out_shape=jax.ShapeDtypeStruct((n, out_ft), jnp.float32),
        scratch_shapes=[
            pltpu.VMEM((n, out_ft), jnp.bfloat16),
            pltpu.VMEM((_BI, n), jnp.bfloat16),
            pltpu.VMEM((_BI, n), jnp.bfloat16),
            pltpu.VMEM((1, out_ft), jnp.float32),
            pltpu.VMEM((1, out_ft), jnp.float32),
        ],
        interpret=interpret,
    )(seq, W, adj, gamma2, beta2, pw2)
    return out


# speedup vs baseline: 1.0701x; 1.0701x over previous
import functools

import jax
import jax.numpy as jnp
from jax.experimental import pallas as pl
from jax.experimental.pallas import tpu as pltpu

_N = 10000
_BI = 200
_NS = 2
_NI = _N // (_BI * _NS)


def _fused_kernel(seq_ref, w_ref, adjl_ref, adjr_ref, gamma_ref, beta_ref,
                  pw_ref, out_ref, x_ref, sum_ref, sq_ref):
    i = pl.program_id(0)

    @pl.when(i == 0)
    def _compute_x():
        x_ref[...] = jax.lax.dot_general(
            seq_ref[...], w_ref[...],
            dimension_numbers=(((1,), (1,)), ((), ())),
            preferred_element_type=jnp.float32,
        )

    ya = jnp.dot(adjl_ref[...], x_ref[...],
                 preferred_element_type=jnp.float32)
    yb = jnp.dot(adjr_ref[...], x_ref[...],
                 preferred_element_type=jnp.float32)
    base = i * _BI * _NS
    out_ref[pl.ds(base, _BI), :] = ya
    out_ref[pl.ds(base + _BI, _BI), :] = yb

    s = jnp.sum(ya, axis=0, keepdims=True) + jnp.sum(yb, axis=0, keepdims=True)
    q = (jnp.sum(ya * ya, axis=0, keepdims=True)
         + jnp.sum(yb * yb, axis=0, keepdims=True))

    @pl.when(i == 0)
    def _init_stats():
        sum_ref[...] = s
        sq_ref[...] = q

    @pl.when(i > 0)
    def _acc_stats():
        sum_ref[...] += s
        sq_ref[...] += q

    @pl.when(i == _NI - 1)
    def _epilogue():
        mean = sum_ref[...] / _N
        var = sq_ref[...] / _N - mean * mean
        scale = gamma_ref[...] / jnp.sqrt(var + 1e-5)
        o = (out_ref[...] - mean) * scale + beta_ref[...]
        out_ref[...] = jnp.where(o >= 0, o, pw_ref[...] * o)


@functools.partial(jax.jit, static_argnames=("interpret",))
def kernel(seq, adj, W, bias, gamma, beta, prelu_w, interpret=False):
    n, in_ft = seq.shape
    out_ft = W.shape[0]

    gamma2 = gamma.reshape(1, out_ft)
    beta2 = beta.reshape(1, out_ft)
    pw2 = jnp.broadcast_to(prelu_w.reshape(1, -1), (1, out_ft))
    del bias

    const = lambda i: (0, 0)
    out = pl.pallas_call(
        _fused_kernel,
        grid=(_NI,),
        in_specs=[
            pl.BlockSpec((n, in_ft), const),
            pl.BlockSpec((out_ft, in_ft), const),
            pl.BlockSpec((_BI, n), lambda i: (2 * i, 0)),
            pl.BlockSpec((_BI, n), lambda i: (2 * i + 1, 0)),
            pl.BlockSpec((1, out_ft), const),
            pl.BlockSpec((1, out_ft), const),
            pl.BlockSpec((1, out_ft), const),
        ],
        out_specs=pl.BlockSpec((n, out_ft), const),
        out_shape=jax.ShapeDtypeStruct((n, out_ft), jnp.float32),
        scratch_shapes=[
            pltpu.VMEM((n, out_ft), jnp.float32),
            pltpu.VMEM((1, out_ft), jnp.float32),
            pltpu.VMEM((1, out_ft), jnp.float32),
        ],
        interpret=interpret,
    )(seq, W, adj, adj, gamma2, beta2, pw2)
    return out


# speedup vs baseline: 1.0929x; 1.0213x over previous
import functools

import jax
import jax.numpy as jnp
from jax.experimental import pallas as pl
from jax.experimental.pallas import tpu as pltpu

_N = 10000
_BI = 200
_NI = _N // _BI
_NBUF = 3


def _fused_kernel(seq_ref, w_ref, adj_ref, gamma_ref, beta_ref, pw_ref,
                  out_ref, x_ref, abuf_ref, sum_ref, sq_ref, sems):
    i = pl.program_id(0)

    def copy_in(blk, slot):
        pltpu.make_async_copy(
            adj_ref.at[pl.ds(blk * _BI, _BI), :],
            abuf_ref.at[slot],
            sems.at[slot],
        ).start()

    @pl.when(i == 0)
    def _warmup():
        for d in range(_NBUF - 1):
            copy_in(d, d)
        x_ref[...] = jax.lax.dot_general(
            seq_ref[...], w_ref[...],
            dimension_numbers=(((1,), (1,)), ((), ())),
            preferred_element_type=jnp.float32,
        )

    nxt = i + _NBUF - 1

    @pl.when(nxt < _NI)
    def _prefetch():
        copy_in(nxt, jax.lax.rem(nxt, _NBUF))

    slot = jax.lax.rem(i, _NBUF)
    pltpu.make_async_copy(
        adj_ref.at[pl.ds(i * _BI, _BI), :],
        abuf_ref.at[slot],
        sems.at[slot],
    ).wait()

    y_blk = jnp.dot(abuf_ref[slot], x_ref[...],
                    preferred_element_type=jnp.float32)
    out_ref[pl.ds(i * _BI, _BI), :] = y_blk

    s = jnp.sum(y_blk, axis=0, keepdims=True)
    q = jnp.sum(y_blk * y_blk, axis=0, keepdims=True)

    @pl.when(i == 0)
    def _init_stats():
        sum_ref[...] = s
        sq_ref[...] = q

    @pl.when(i > 0)
    def _acc_stats():
        sum_ref[...] += s
        sq_ref[...] += q

    @pl.when(i == _NI - 1)
    def _epilogue():
        mean = sum_ref[...] / _N
        var = sq_ref[...] / _N - mean * mean
        scale = gamma_ref[...] / jnp.sqrt(var + 1e-5)
        o = (out_ref[...] - mean) * scale + beta_ref[...]
        out_ref[...] = jnp.where(o >= 0, o, pw_ref[...] * o)


@functools.partial(jax.jit, static_argnames=("interpret",))
def kernel(seq, adj, W, bias, gamma, beta, prelu_w, interpret=False):
    n, in_ft = seq.shape
    out_ft = W.shape[0]

    gamma2 = gamma.reshape(1, out_ft)
    beta2 = beta.reshape(1, out_ft)
    pw2 = jnp.broadcast_to(prelu_w.reshape(1, -1), (1, out_ft))
    del bias

    const = lambda i: (0, 0)
    out = pl.pallas_call(
        _fused_kernel,
        grid=(_NI,),
        in_specs=[
            pl.BlockSpec((n, in_ft), const),
            pl.BlockSpec((out_ft, in_ft), const),
            pl.BlockSpec(memory_space=pltpu.MemorySpace.HBM),
            pl.BlockSpec((1, out_ft), const),
            pl.BlockSpec((1, out_ft), const),
            pl.BlockSpec((1, out_ft), const),
        ],
        out_specs=pl.BlockSpec((n, out_ft), const),
        out_shape=jax.ShapeDtypeStruct((n, out_ft), jnp.float32),
        scratch_shapes=[
            pltpu.VMEM((n, out_ft), jnp.float32),
            pltpu.VMEM((_NBUF, _BI, n), jnp.float32),
            pltpu.VMEM((1, out_ft), jnp.float32),
            pltpu.VMEM((1, out_ft), jnp.float32),
            pltpu.SemaphoreType.DMA((_NBUF,)),
        ],
        interpret=interpret,
    )(seq, W, adj, gamma2, beta2, pw2)
    return out
